# Initial kernel scaffold; baseline (speedup 1.0000x reference)
import jax
import jax.numpy as jnp
from jax.experimental import pallas as pl


def kernel(user, item, user_mf, item_mf, user_mlp, item_mlp, W1, b1, W2, b2, Wp, bp):
    raise NotImplementedError("write your pallas kernel here")


# speedup vs baseline: 1.8062x; 1.8062x over previous
import functools

import jax
import jax.numpy as jnp
from jax import lax
from jax.experimental import pallas as pl
from jax.experimental.pallas import tpu as pltpu
from jax.experimental.pallas import tpu_sc as plsc

BATCH = 16384
EDIM = 64
NC = 2
NS = 16
NW = NC * NS
BPW = BATCH // NW
CHUNK = 128
NCH = BPW // CHUNK

_f32 = jnp.float32


def _sc_gather_body(user_hbm, item_hbm, umf_hbm, imf_hbm, umlp_hbm, imlp_hbm,
                    out_umf, out_imf, out_umlp, out_imlp,
                    idx_u, idx_i, bufa, bufb, sem):
  wid = lax.axis_index("s") * NC + lax.axis_index("c")
  r0 = wid * NCH
  pltpu.sync_copy(user_hbm.at[pl.ds(r0, NCH)], idx_u)
  pltpu.sync_copy(item_hbm.at[pl.ds(r0, NCH)], idx_i)

  def gather_pair(tab_u, tab_i, dst_u, dst_i):
    copies = []
    for j in range(NCH):
      copies.append(pltpu.async_copy(tab_u.at[idx_u.at[j]], bufa.at[j], sem))
      copies.append(pltpu.async_copy(tab_i.at[idx_i.at[j]], bufb.at[j], sem))
    for c in copies:
      c.wait()
    pltpu.sync_copy(bufa, dst_u.at[pl.ds(r0, NCH)])
    pltpu.sync_copy(bufb, dst_i.at[pl.ds(r0, NCH)])

  gather_pair(umf_hbm, imf_hbm, out_umf, out_imf)
  gather_pair(umlp_hbm, imlp_hbm, out_umlp, out_imlp)


def _sc_gather(user2d, item2d, user_mf, item_mf, user_mlp, item_mlp):
  mesh = plsc.VectorSubcoreMesh(core_axis_name="c", subcore_axis_name="s")
  out3 = jax.ShapeDtypeStruct((BATCH // CHUNK, CHUNK, EDIM), _f32)
  fn = functools.partial(
      pl.kernel,
      mesh=mesh,
      out_type=[out3, out3, out3, out3],
      scratch_types=[
          pltpu.VMEM((NCH, CHUNK), jnp.int32),
          pltpu.VMEM((NCH, CHUNK), jnp.int32),
          pltpu.VMEM((NCH, CHUNK, EDIM), _f32),
          pltpu.VMEM((NCH, CHUNK, EDIM), _f32),
          pltpu.SemaphoreType.DMA,
      ],
      compiler_params=pltpu.CompilerParams(use_tc_tiling_on_sc=False),
  )(_sc_gather_body)
  return fn(user2d, item2d, user_mf, item_mf, user_mlp, item_mlp)


def _tc_body(ume, ime, umlp, imlp, w1, b1, w2, b2, wpm, wph, bp, out):
  mf = ume[...] * ime[...]
  x = jnp.concatenate([umlp[...], imlp[...]], axis=1)
  h1 = lax.dot_general(x, w1[...], (((1,), (1,)), ((), ())),
                       preferred_element_type=_f32)
  h1 = jnp.maximum(h1 + b1[...], 0.0)
  h2 = lax.dot_general(h1, w2[...], (((1,), (1,)), ((), ())),
                       preferred_element_type=_f32)
  h2 = jnp.maximum(h2 + b2[...], 0.0)
  logit = (jnp.sum(mf * wpm[...], axis=1, keepdims=True)
           + jnp.sum(h2 * wph[...], axis=1, keepdims=True)
           + bp[...])
  out[...] = jax.nn.sigmoid(logit).reshape(out.shape)


def _tc_mlp(ume, ime, umlp, imlp, W1, b1, W2, b2, wpm, wph, bp):
  blk = 2048
  grid = BATCH // blk
  row_spec = pl.BlockSpec((blk, EDIM), lambda i: (i, 0))
  full = lambda shape: pl.BlockSpec(shape, lambda i: (0, 0))
  out2 = pl.pallas_call(
      _tc_body,
      grid=(grid,),
      in_specs=[row_spec, row_spec, row_spec, row_spec,
                full((128, 128)), full((1, 128)),
                full((64, 128)), full((1, 64)),
                full((1, 64)), full((1, 64)), full((1, 1))],
      out_specs=pl.BlockSpec((1, 1, blk), lambda i: (i, 0, 0)),
      out_shape=jax.ShapeDtypeStruct((grid, 1, blk), _f32),
  )(ume, ime, umlp, imlp, W1, b1, W2, b2, wpm, wph, bp)
  return out2.reshape(BATCH)


def kernel(user, item, user_mf, item_mf, user_mlp, item_mlp,
           W1, b1, W2, b2, Wp, bp):
  user2d = user.astype(jnp.int32).reshape(BATCH // CHUNK, CHUNK)
  item2d = item.astype(jnp.int32).reshape(BATCH // CHUNK, CHUNK)
  ume3, ime3, umlpe3, imlpe3 = _sc_gather(
      user2d, item2d, user_mf, item_mf, user_mlp, item_mlp)
  ume = ume3.reshape(BATCH, EDIM)
  ime = ime3.reshape(BATCH, EDIM)
  umlpe = umlpe3.reshape(BATCH, EDIM)
  imlpe = imlpe3.reshape(BATCH, EDIM)
  wp = Wp.reshape(128)
  wpm = wp[:EDIM].reshape(1, EDIM)
  wph = wp[EDIM:].reshape(1, EDIM)
  return _tc_mlp(ume, ime, umlpe, imlpe,
                 W1, b1.reshape(1, 128), W2, b2.reshape(1, 64),
                 wpm, wph, bp.reshape(1, 1))


# speedup vs baseline: 2.2231x; 1.2308x over previous
import functools

import jax
import jax.numpy as jnp
from jax import lax
from jax.experimental import pallas as pl
from jax.experimental.pallas import tpu as pltpu
from jax.experimental.pallas import tpu_sc as plsc

BATCH = 16384
EDIM = 64
ROW = 2 * EDIM
NC = 2
NS = 16
NW = NC * NS
BPW = BATCH // NW
CHUNK = 128
NCH = BPW // CHUNK

_f32 = jnp.float32


def _sc_gather_body(user_hbm, item_hbm, utab, itab, out_u, out_i,
                    idx_u, idx_i, buf, sem):
  wid = lax.axis_index("s") * NC + lax.axis_index("c")
  r0 = wid * NCH
  pltpu.sync_copy(user_hbm.at[pl.ds(r0, NCH)], idx_u)
  pltpu.sync_copy(item_hbm.at[pl.ds(r0, NCH)], idx_i)

  def gather(tab, idx, dst):
    copies = []
    for j in range(NCH):
      copies.append(pltpu.async_copy(tab.at[idx.at[j]], buf.at[j], sem))
    for c in copies:
      c.wait()
    pltpu.sync_copy(buf, dst.at[pl.ds(r0, NCH)])

  gather(utab, idx_u, out_u)
  gather(itab, idx_i, out_i)


def _sc_gather(user2d, item2d, utab, itab):
  mesh = plsc.VectorSubcoreMesh(core_axis_name="c", subcore_axis_name="s")
  out3 = jax.ShapeDtypeStruct((BATCH // CHUNK, CHUNK, ROW), _f32)
  fn = functools.partial(
      pl.kernel,
      mesh=mesh,
      out_type=[out3, out3],
      scratch_types=[
          pltpu.VMEM((NCH, CHUNK), jnp.int32),
          pltpu.VMEM((NCH, CHUNK), jnp.int32),
          pltpu.VMEM((NCH, CHUNK, ROW), _f32),
          pltpu.SemaphoreType.DMA,
      ],
      compiler_params=pltpu.CompilerParams(use_tc_tiling_on_sc=False),
  )(_sc_gather_body)
  return fn(user2d, item2d, utab, itab)


def _pack_body(umf, umlp, imf, imlp, out_u, out_i):
  out_u[:, :EDIM] = umf[...]
  out_u[:, EDIM:] = umlp[...]
  out_i[:, :EDIM] = imf[...]
  out_i[:, EDIM:] = imlp[...]


def _pack_tables(user_mf, user_mlp, item_mf, item_mlp):
  v = user_mf.shape[0]
  tab = jax.ShapeDtypeStruct((v, ROW), _f32)
  return pl.pallas_call(
      _pack_body,
      out_shape=[tab, tab],
  )(user_mf, user_mlp, item_mf, item_mlp)


def _tc_body(uref, iref, w1, b1, w2, b2, wpm, wph, bp, out):
  u = uref[...]
  i = iref[...]
  mf = u[:, :EDIM] * i[:, :EDIM]
  h1 = (lax.dot_general(u[:, EDIM:], w1[:, :EDIM], (((1,), (1,)), ((), ())),
                        preferred_element_type=_f32)
        + lax.dot_general(i[:, EDIM:], w1[:, EDIM:], (((1,), (1,)), ((), ())),
                          preferred_element_type=_f32))
  h1 = jnp.maximum(h1 + b1[...], 0.0)
  h2 = lax.dot_general(h1, w2[...], (((1,), (1,)), ((), ())),
                       preferred_element_type=_f32)
  h2 = jnp.maximum(h2 + b2[...], 0.0)
  logit = (jnp.sum(mf * wpm[...], axis=1, keepdims=True)
           + jnp.sum(h2 * wph[...], axis=1, keepdims=True)
           + bp[...])
  out[...] = jax.nn.sigmoid(logit).reshape(out.shape)


def _tc_mlp(urows, irows, W1, b1, W2, b2, wpm, wph, bp):
  blk = 2048
  grid = BATCH // blk
  row_spec = pl.BlockSpec((blk, ROW), lambda i: (i, 0))
  full = lambda shape: pl.BlockSpec(shape, lambda i: (0, 0))
  out2 = pl.pallas_call(
      _tc_body,
      grid=(grid,),
      in_specs=[row_spec, row_spec,
                full((128, 128)), full((1, 128)),
                full((64, 128)), full((1, 64)),
                full((1, 64)), full((1, 64)), full((1, 1))],
      out_specs=pl.BlockSpec((1, 1, blk), lambda i: (i, 0, 0)),
      out_shape=jax.ShapeDtypeStruct((grid, 1, blk), _f32),
  )(urows, irows, W1, b1, W2, b2, wpm, wph, bp)
  return out2.reshape(BATCH)


def kernel(user, item, user_mf, item_mf, user_mlp, item_mlp,
           W1, b1, W2, b2, Wp, bp):
  user2d = user.astype(jnp.int32).reshape(BATCH // CHUNK, CHUNK)
  item2d = item.astype(jnp.int32).reshape(BATCH // CHUNK, CHUNK)
  utab, itab = _pack_tables(user_mf, user_mlp, item_mf, item_mlp)
  urows3, irows3 = _sc_gather(user2d, item2d, utab, itab)
  urows = urows3.reshape(BATCH, ROW)
  irows = irows3.reshape(BATCH, ROW)
  wp = Wp.reshape(128)
  wpm = wp[:EDIM].reshape(1, EDIM)
  wph = wp[EDIM:].reshape(1, EDIM)
  return _tc_mlp(urows, irows,
                 W1, b1.reshape(1, 128), W2, b2.reshape(1, 64),
                 wpm, wph, bp.reshape(1, 1))


# speedup vs baseline: 2.7322x; 1.2290x over previous
import functools

import jax
import jax.numpy as jnp
from jax import lax
from jax.experimental import pallas as pl
from jax.experimental.pallas import tpu as pltpu
from jax.experimental.pallas import tpu_sc as plsc

BATCH = 16384
EDIM = 64
ROW = 2 * EDIM
NC = 2
NS = 16
NW = NC * NS
BPW = BATCH // NW
CHUNK = 128
NCH = BPW // CHUNK

_f32 = jnp.float32


def _sc_gather_body(user_hbm, item_hbm, utab, itab, out_u, out_i,
                    idx_u, idx_i, buf, sem):
  wid = lax.axis_index("s") * NC + lax.axis_index("c")
  r0 = wid * NCH
  pltpu.sync_copy(user_hbm.at[pl.ds(r0, NCH)], idx_u)
  pltpu.sync_copy(item_hbm.at[pl.ds(r0, NCH)], idx_i)

  def gather(tab, idx, dst):
    copies = []
    for j in range(NCH):
      copies.append(pltpu.async_copy(tab.at[idx.at[j]], buf.at[j], sem))
    for c in copies:
      c.wait()
    pltpu.sync_copy(buf, dst.at[pl.ds(r0, NCH)])

  gather(utab, idx_u, out_u)
  gather(itab, idx_i, out_i)


def _sc_gather(user2d, item2d, utab, itab):
  mesh = plsc.VectorSubcoreMesh(core_axis_name="c", subcore_axis_name="s")
  out3 = jax.ShapeDtypeStruct((BATCH // CHUNK, CHUNK, ROW), _f32)
  fn = functools.partial(
      pl.kernel,
      mesh=mesh,
      out_type=[out3, out3],
      scratch_types=[
          pltpu.VMEM((NCH, CHUNK), jnp.int32),
          pltpu.VMEM((NCH, CHUNK), jnp.int32),
          pltpu.VMEM((NCH, CHUNK, ROW), _f32),
          pltpu.SemaphoreType.DMA,
      ],
      compiler_params=pltpu.CompilerParams(use_tc_tiling_on_sc=False),
  )(_sc_gather_body)
  return fn(user2d, item2d, utab, itab)


def _pack_body(umf_t, umlp_t, imf_t, imlp_t, out_u, out_i):
  out_u[:, :EDIM] = umf_t[...].T
  out_u[:, EDIM:] = umlp_t[...].T
  out_i[:, :EDIM] = imf_t[...].T
  out_i[:, EDIM:] = imlp_t[...].T


def _pack_tables(user_mf, user_mlp, item_mf, item_mlp):
  v = user_mf.shape[0]
  tab = jax.ShapeDtypeStruct((v, ROW), _f32)
  return pl.pallas_call(
      _pack_body,
      out_shape=[tab, tab],
  )(user_mf.T, user_mlp.T, item_mf.T, item_mlp.T)


def _tc_body(uref, iref, w1, b1, w2, b2, wpm, wph, bp, out):
  u = uref[...]
  i = iref[...]
  mf = u[:, :EDIM] * i[:, :EDIM]
  h1 = (lax.dot_general(u[:, EDIM:], w1[:, :EDIM], (((1,), (1,)), ((), ())),
                        preferred_element_type=_f32)
        + lax.dot_general(i[:, EDIM:], w1[:, EDIM:], (((1,), (1,)), ((), ())),
                          preferred_element_type=_f32))
  h1 = jnp.maximum(h1 + b1[...], 0.0)
  h2 = lax.dot_general(h1, w2[...], (((1,), (1,)), ((), ())),
                       preferred_element_type=_f32)
  h2 = jnp.maximum(h2 + b2[...], 0.0)
  logit = (jnp.sum(mf * wpm[...], axis=1, keepdims=True)
           + jnp.sum(h2 * wph[...], axis=1, keepdims=True)
           + bp[...])
  out[...] = jax.nn.sigmoid(logit).reshape(out.shape)


def _tc_mlp(urows, irows, W1, b1, W2, b2, wpm, wph, bp):
  blk = 2048
  grid = BATCH // blk
  row_spec = pl.BlockSpec((blk, ROW), lambda i: (i, 0))
  full = lambda shape: pl.BlockSpec(shape, lambda i: (0, 0))
  out2 = pl.pallas_call(
      _tc_body,
      grid=(grid,),
      in_specs=[row_spec, row_spec,
                full((128, 128)), full((1, 128)),
                full((64, 128)), full((1, 64)),
                full((1, 64)), full((1, 64)), full((1, 1))],
      out_specs=pl.BlockSpec((1, 1, blk), lambda i: (i, 0, 0)),
      out_shape=jax.ShapeDtypeStruct((grid, 1, blk), _f32),
  )(urows, irows, W1, b1, W2, b2, wpm, wph, bp)
  return out2.reshape(BATCH)


def kernel(user, item, user_mf, item_mf, user_mlp, item_mlp,
           W1, b1, W2, b2, Wp, bp):
  user2d = user.astype(jnp.int32).reshape(BATCH // CHUNK, CHUNK)
  item2d = item.astype(jnp.int32).reshape(BATCH // CHUNK, CHUNK)
  utab, itab = _pack_tables(user_mf, user_mlp, item_mf, item_mlp)
  urows3, irows3 = _sc_gather(user2d, item2d, utab, itab)
  urows = urows3.reshape(BATCH, ROW)
  irows = irows3.reshape(BATCH, ROW)
  wp = Wp.reshape(128)
  wpm = wp[:EDIM].reshape(1, EDIM)
  wph = wp[EDIM:].reshape(1, EDIM)
  return _tc_mlp(urows, irows,
                 W1, b1.reshape(1, 128), W2, b2.reshape(1, 64),
                 wpm, wph, bp.reshape(1, 1))
